# Initial kernel scaffold; baseline (speedup 1.0000x reference)
import jax
import jax.numpy as jnp
from jax.experimental import pallas as pl


def kernel(x, edge_index, W1, b1, W2, b2):
    raise NotImplementedError("write your pallas kernel here")


# speedup vs baseline: 7.7993x; 7.7993x over previous
import functools

import jax
import jax.numpy as jnp
from jax import lax
from jax.experimental import pallas as pl
from jax.experimental.pallas import tpu as pltpu
from jax.experimental.pallas import tpu_sc as plsc

N = 10000
E = 320000
D = 128
NC = 2
NS = 16
NW = NC * NS
C = 128
NCH = 80
EPW = NCH * C
EP = NW * EPW
GRP = 8
NGRP = NCH // GRP
NP = N + 8
SROWS = 624
TAIL = N - NS * SROWS
BM = 1000

_PREC = jax.lax.Precision.HIGHEST


def _zero_fill(buf, nlanes):
    @pl.loop(0, buf.shape[0])
    def _(i):
        @pl.loop(0, nlanes // 16)
        def _(j):
            buf[i, pl.ds(j * 16, 16)] = jnp.zeros((16,), jnp.float32)


def _zero_acc(buf, acc_sh, s):
    base = s * SROWS
    for t in range(4):
        pltpu.sync_copy(buf, acc_sh.at[pl.ds(base + t * 128, 128)])
    pltpu.sync_copy(buf.at[pl.ds(0, SROWS - 512)],
                    acc_sh.at[pl.ds(base + 512, SROWS - 512)])

    @pl.when(s == NS - 1)
    def _():
        pltpu.sync_copy(buf.at[pl.ds(0, TAIL)],
                        acc_sh.at[pl.ds(NS * SROWS, TAIL)])


def _write_back(acc_sh, out_hbm, c, s):
    pltpu.sync_copy(acc_sh.at[pl.ds(s * SROWS, SROWS)],
                    out_hbm.at[pl.ds(c * N + s * SROWS, SROWS)])

    @pl.when(s == NS - 1)
    def _():
        pltpu.sync_copy(acc_sh.at[pl.ds(NS * SROWS, TAIL)],
                        out_hbm.at[pl.ds(c * N + NS * SROWS, TAIL)])


@functools.cache
def _sc_degree_kernel():
    return functools.partial(
        pl.kernel,
        out_type=jax.ShapeDtypeStruct((NC * N, 16), jnp.float32),
        mesh=plsc.VectorSubcoreMesh(core_axis_name="c", subcore_axis_name="s"),
        scratch_types=[
            pltpu.VMEM((GRP, C), jnp.int32),
            pltpu.VMEM((C, 16), jnp.float32),
            pltpu.VMEM((128, 16), jnp.float32),
            pltpu.VMEM_SHARED((NP, 16), jnp.float32),
        ],
    )(_sc_degree_body)


def _sc_degree(col):
    return _sc_degree_kernel()(col)


def _sc_degree_body(col_hbm, out_hbm, col_v, ones_v, z_v, acc_sh):
    c = lax.axis_index("c")
    s = lax.axis_index("s")
    w = s * NC + c

    @pl.loop(0, C)
    def _(i):
        ones_v[i, :] = jnp.ones((16,), jnp.float32)

    _zero_fill(z_v, 16)
    _zero_acc(z_v, acc_sh, s)
    plsc.subcore_barrier()

    @pl.loop(0, NGRP)
    def _(g):
        pltpu.sync_copy(col_hbm.at[w].at[pl.ds(g * GRP, GRP)], col_v)

        @pl.loop(0, GRP)
        def _(j):
            pltpu.sync_copy(ones_v, acc_sh.at[col_v.at[j]], add=True)

    plsc.subcore_barrier()
    _write_back(acc_sh, out_hbm, c, s)


@functools.cache
def _sc_scatter_kernel():
    return functools.partial(
        pl.kernel,
        out_type=jax.ShapeDtypeStruct((NC * N, D), jnp.float32),
        mesh=plsc.VectorSubcoreMesh(core_axis_name="c", subcore_axis_name="s"),
        scratch_types=[
            pltpu.VMEM((GRP, C), jnp.int32),
            pltpu.VMEM((GRP, C), jnp.int32),
            pltpu.VMEM((C, D), jnp.float32),
            pltpu.VMEM_SHARED((NP, D), jnp.float32),
            pltpu.SemaphoreType.DMA,
        ],
    )(_sc_scatter_body)


def _sc_scatter(y, row, col):
    return _sc_scatter_kernel()(y, row, col)


def _sc_scatter_body(y_hbm, row_hbm, col_hbm, out_hbm,
                     row_v, col_v, buf, acc_sh, sem):
    c = lax.axis_index("c")
    s = lax.axis_index("s")
    w = s * NC + c

    _zero_fill(buf, D)
    _zero_acc(buf, acc_sh, s)
    plsc.subcore_barrier()

    @pl.loop(0, NGRP)
    def _(g):
        pltpu.sync_copy(row_hbm.at[w].at[pl.ds(g * GRP, GRP)], row_v)
        pltpu.sync_copy(col_hbm.at[w].at[pl.ds(g * GRP, GRP)], col_v)

        @pl.loop(0, GRP)
        def _(j):
            pltpu.async_copy(y_hbm.at[row_v.at[j]], buf, sem).wait()
            pltpu.sync_copy(buf, acc_sh.at[col_v.at[j]], add=True)

    plsc.subcore_barrier()
    _write_back(acc_sh, out_hbm, c, s)


def _mm_body(x_ref, w_ref, o_ref):
    o_ref[...] = jnp.dot(x_ref[...], w_ref[...], precision=_PREC,
                         preferred_element_type=jnp.float32)


def _tc_matmul(x, W):
    return pl.pallas_call(
        _mm_body,
        grid=(N // BM,),
        in_specs=[
            pl.BlockSpec((BM, D), lambda i: (i, 0)),
            pl.BlockSpec((D, D), lambda i: (0, 0)),
        ],
        out_specs=pl.BlockSpec((BM, D), lambda i: (i, 0)),
        out_shape=jax.ShapeDtypeStruct((N, D), jnp.float32),
    )(x, W)


def _dinv_of(dp_ref):
    deg = dp_ref[0, :, 0:1] + dp_ref[1, :, 0:1] + 1.0
    return jax.lax.rsqrt(deg)


def _norm_body(dp_ref, xw_ref, y_ref):
    y_ref[...] = xw_ref[...] * _dinv_of(dp_ref)


def _tc_norm(dp, xw):
    return pl.pallas_call(
        _norm_body,
        grid=(N // BM,),
        in_specs=[
            pl.BlockSpec((2, BM, 16), lambda i: (0, i, 0)),
            pl.BlockSpec((BM, D), lambda i: (i, 0)),
        ],
        out_specs=pl.BlockSpec((BM, D), lambda i: (i, 0)),
        out_shape=jax.ShapeDtypeStruct((N, D), jnp.float32),
    )(dp, xw)


def _l1_body(x_ref, dp_ref, s_ref, y1_ref, b_ref, w_ref, h_ref, y2_ref):
    dinv = _dinv_of(dp_ref)
    agg = s_ref[0] + s_ref[1] + y1_ref[...]
    h = jnp.maximum(x_ref[...] + dinv * agg + b_ref[...], 0.0)
    h_ref[...] = h
    y2_ref[...] = jnp.dot(h, w_ref[...], precision=_PREC,
                          preferred_element_type=jnp.float32) * dinv


def _tc_layer1(x, dp, s1, y1, b1, W2):
    return pl.pallas_call(
        _l1_body,
        grid=(N // BM,),
        in_specs=[
            pl.BlockSpec((BM, D), lambda i: (i, 0)),
            pl.BlockSpec((2, BM, 16), lambda i: (0, i, 0)),
            pl.BlockSpec((2, BM, D), lambda i: (0, i, 0)),
            pl.BlockSpec((BM, D), lambda i: (i, 0)),
            pl.BlockSpec((1, D), lambda i: (0, 0)),
            pl.BlockSpec((D, D), lambda i: (0, 0)),
        ],
        out_specs=[
            pl.BlockSpec((BM, D), lambda i: (i, 0)),
            pl.BlockSpec((BM, D), lambda i: (i, 0)),
        ],
        out_shape=[
            jax.ShapeDtypeStruct((N, D), jnp.float32),
            jax.ShapeDtypeStruct((N, D), jnp.float32),
        ],
    )(x, dp, s1, y1, b1, W2)


def _l2_body(h_ref, dp_ref, s_ref, y2_ref, b_ref, o_ref):
    dinv = _dinv_of(dp_ref)
    agg = s_ref[0] + s_ref[1] + y2_ref[...]
    o_ref[...] = h_ref[...] + dinv * agg + b_ref[...]


def _tc_layer2(h, dp, s2, y2, b2):
    return pl.pallas_call(
        _l2_body,
        grid=(N // BM,),
        in_specs=[
            pl.BlockSpec((BM, D), lambda i: (i, 0)),
            pl.BlockSpec((2, BM, 16), lambda i: (0, i, 0)),
            pl.BlockSpec((2, BM, D), lambda i: (0, i, 0)),
            pl.BlockSpec((BM, D), lambda i: (i, 0)),
            pl.BlockSpec((1, D), lambda i: (0, 0)),
        ],
        out_specs=pl.BlockSpec((BM, D), lambda i: (i, 0)),
        out_shape=jax.ShapeDtypeStruct((N, D), jnp.float32),
    )(h, dp, s2, y2, b2)


def kernel(x, edge_index, W1, b1, W2, b2):
    ei = edge_index.astype(jnp.int32)
    pad = EP - E
    row = jnp.concatenate(
        [ei[0], jnp.zeros((pad,), jnp.int32)]).reshape(NW, NCH, C)
    col = jnp.concatenate(
        [ei[1], jnp.full((pad,), N, jnp.int32)]).reshape(NW, NCH, C)
    b1r = b1.reshape(1, D)
    b2r = b2.reshape(1, D)

    dp = _sc_degree(col).reshape(NC, N, 16)
    xw1 = _tc_matmul(x, W1)
    y1 = _tc_norm(dp, xw1)
    s1 = _sc_scatter(y1, row, col).reshape(NC, N, D)
    h, y2 = _tc_layer1(x, dp, s1, y1, b1r, W2)
    s2 = _sc_scatter(y2, row, col).reshape(NC, N, D)
    return _tc_layer2(h, dp, s2, y2, b2r)


# speedup vs baseline: 8.9435x; 1.1467x over previous
import functools

import jax
import jax.numpy as jnp
from jax import lax
from jax.experimental import pallas as pl
from jax.experimental.pallas import tpu as pltpu
from jax.experimental.pallas import tpu_sc as plsc

N = 10000
E = 320000
D = 128
NC = 2
NS = 16
NW = NC * NS
C = 128
NCH = 80
EPW = NCH * C
EP = NW * EPW
GRP = 8
NGRP = NCH // GRP
NP = N + 8
SROWS = 624
TAIL = N - NS * SROWS
BM = 1000

_PREC = jax.lax.Precision.HIGHEST


def _zero_fill(buf, nlanes):
    @pl.loop(0, buf.shape[0])
    def _(i):
        @pl.loop(0, nlanes // 16)
        def _(j):
            buf[i, pl.ds(j * 16, 16)] = jnp.zeros((16,), jnp.float32)


def _zero_acc(buf, acc_sh, s):
    base = s * SROWS
    for t in range(4):
        pltpu.sync_copy(buf, acc_sh.at[pl.ds(base + t * 128, 128)])
    pltpu.sync_copy(buf.at[pl.ds(0, SROWS - 512)],
                    acc_sh.at[pl.ds(base + 512, SROWS - 512)])

    @pl.when(s == NS - 1)
    def _():
        pltpu.sync_copy(buf.at[pl.ds(0, TAIL)],
                        acc_sh.at[pl.ds(NS * SROWS, TAIL)])


def _write_back(acc_sh, out_hbm, c, s):
    pltpu.sync_copy(acc_sh.at[pl.ds(s * SROWS, SROWS)],
                    out_hbm.at[pl.ds(c * N + s * SROWS, SROWS)])

    @pl.when(s == NS - 1)
    def _():
        pltpu.sync_copy(acc_sh.at[pl.ds(NS * SROWS, TAIL)],
                        out_hbm.at[pl.ds(c * N + NS * SROWS, TAIL)])


@functools.cache
def _sc_degree_kernel():
    return functools.partial(
        pl.kernel,
        out_type=jax.ShapeDtypeStruct((NC * N, 16), jnp.float32),
        mesh=plsc.VectorSubcoreMesh(core_axis_name="c", subcore_axis_name="s"),
        scratch_types=[
            pltpu.VMEM((GRP, C), jnp.int32),
            pltpu.VMEM((C, 16), jnp.float32),
            pltpu.VMEM((128, 16), jnp.float32),
            pltpu.VMEM_SHARED((NP, 16), jnp.float32),
        ],
    )(_sc_degree_body)


def _sc_degree(col):
    return _sc_degree_kernel()(col)


def _sc_degree_body(col_hbm, out_hbm, col_v, ones_v, z_v, acc_sh):
    c = lax.axis_index("c")
    s = lax.axis_index("s")
    w = s * NC + c

    @pl.loop(0, C)
    def _(i):
        ones_v[i, :] = jnp.ones((16,), jnp.float32)

    _zero_fill(z_v, 16)
    _zero_acc(z_v, acc_sh, s)
    plsc.subcore_barrier()

    @pl.loop(0, NGRP)
    def _(g):
        pltpu.sync_copy(col_hbm.at[w].at[pl.ds(g * GRP, GRP)], col_v)

        @pl.loop(0, GRP)
        def _(j):
            pltpu.sync_copy(ones_v, acc_sh.at[col_v.at[j]], add=True)

    plsc.subcore_barrier()
    _write_back(acc_sh, out_hbm, c, s)


HCH = NCH // 2


@functools.cache
def _sc_scatter_kernel():
    return functools.partial(
        pl.kernel,
        out_type=jax.ShapeDtypeStruct((NC * N, D), jnp.float32),
        mesh=plsc.VectorSubcoreMesh(core_axis_name="c", subcore_axis_name="s"),
        scratch_types=[
            pltpu.VMEM((HCH, C), jnp.int32),
            pltpu.VMEM((HCH, C), jnp.int32),
            pltpu.VMEM((C, D), jnp.float32),
            pltpu.VMEM((C, D), jnp.float32),
            pltpu.VMEM_SHARED((NP, D), jnp.float32),
            pltpu.SemaphoreType.DMA,
            pltpu.SemaphoreType.DMA,
            pltpu.SemaphoreType.DMA,
            pltpu.SemaphoreType.DMA,
        ],
    )(_sc_scatter_body)


def _sc_scatter(y, row, col):
    return _sc_scatter_kernel()(y, row, col)


def _sc_scatter_body(y_hbm, row_hbm, col_hbm, out_hbm,
                     row_v, col_v, buf0, buf1, acc_sh, gs0, gs1, ss0, ss1):
    c = lax.axis_index("c")
    s = lax.axis_index("s")
    w = s * NC + c

    _zero_fill(buf0, D)
    _zero_acc(buf0, acc_sh, s)
    plsc.subcore_barrier()

    def gather(j, buf, sem):
        pltpu.async_copy(y_hbm.at[row_v.at[j]], buf, sem)

    def wait_gather(buf, sem):
        pltpu.make_async_copy(y_hbm.at[row_v.at[0]], buf, sem).wait()

    def scat(j, buf, sem):
        pltpu.async_copy(buf, acc_sh.at[col_v.at[j]], sem, add=True)

    def wait_scat(buf, sem):
        pltpu.make_async_copy(buf, acc_sh.at[col_v.at[0]], sem).wait()

    for h in range(2):
        pltpu.sync_copy(row_hbm.at[w].at[pl.ds(h * HCH, HCH)], row_v)
        pltpu.sync_copy(col_hbm.at[w].at[pl.ds(h * HCH, HCH)], col_v)

        gather(0, buf0, gs0)
        gather(1, buf1, gs1)
        wait_gather(buf0, gs0)
        scat(0, buf0, ss0)

        @pl.loop(1, HCH // 2)
        def _(p):
            a = 2 * p
            wait_scat(buf0, ss0)
            gather(a, buf0, gs0)
            wait_gather(buf1, gs1)
            scat(a - 1, buf1, ss1)
            wait_scat(buf1, ss1)
            gather(a + 1, buf1, gs1)
            wait_gather(buf0, gs0)
            scat(a, buf0, ss0)

        wait_gather(buf1, gs1)
        scat(HCH - 1, buf1, ss1)
        wait_scat(buf1, ss1)
        wait_scat(buf0, ss0)

    plsc.subcore_barrier()
    _write_back(acc_sh, out_hbm, c, s)


def _mm_body(x_ref, w_ref, o_ref):
    o_ref[...] = jnp.dot(x_ref[...], w_ref[...], precision=_PREC,
                         preferred_element_type=jnp.float32)


def _tc_matmul(x, W):
    return pl.pallas_call(
        _mm_body,
        grid=(N // BM,),
        in_specs=[
            pl.BlockSpec((BM, D), lambda i: (i, 0)),
            pl.BlockSpec((D, D), lambda i: (0, 0)),
        ],
        out_specs=pl.BlockSpec((BM, D), lambda i: (i, 0)),
        out_shape=jax.ShapeDtypeStruct((N, D), jnp.float32),
    )(x, W)


def _dinv_of(dp_ref):
    deg = dp_ref[0, :, 0:1] + dp_ref[1, :, 0:1] + 1.0
    return jax.lax.rsqrt(deg)


def _norm_body(dp_ref, xw_ref, y_ref):
    y_ref[...] = xw_ref[...] * _dinv_of(dp_ref)


def _tc_norm(dp, xw):
    return pl.pallas_call(
        _norm_body,
        grid=(N // BM,),
        in_specs=[
            pl.BlockSpec((2, BM, 16), lambda i: (0, i, 0)),
            pl.BlockSpec((BM, D), lambda i: (i, 0)),
        ],
        out_specs=pl.BlockSpec((BM, D), lambda i: (i, 0)),
        out_shape=jax.ShapeDtypeStruct((N, D), jnp.float32),
    )(dp, xw)


def _l1_body(x_ref, dp_ref, s_ref, y1_ref, b_ref, w_ref, h_ref, y2_ref):
    dinv = _dinv_of(dp_ref)
    agg = s_ref[0] + s_ref[1] + y1_ref[...]
    h = jnp.maximum(x_ref[...] + dinv * agg + b_ref[...], 0.0)
    h_ref[...] = h
    y2_ref[...] = jnp.dot(h, w_ref[...], precision=_PREC,
                          preferred_element_type=jnp.float32) * dinv


def _tc_layer1(x, dp, s1, y1, b1, W2):
    return pl.pallas_call(
        _l1_body,
        grid=(N // BM,),
        in_specs=[
            pl.BlockSpec((BM, D), lambda i: (i, 0)),
            pl.BlockSpec((2, BM, 16), lambda i: (0, i, 0)),
            pl.BlockSpec((2, BM, D), lambda i: (0, i, 0)),
            pl.BlockSpec((BM, D), lambda i: (i, 0)),
            pl.BlockSpec((1, D), lambda i: (0, 0)),
            pl.BlockSpec((D, D), lambda i: (0, 0)),
        ],
        out_specs=[
            pl.BlockSpec((BM, D), lambda i: (i, 0)),
            pl.BlockSpec((BM, D), lambda i: (i, 0)),
        ],
        out_shape=[
            jax.ShapeDtypeStruct((N, D), jnp.float32),
            jax.ShapeDtypeStruct((N, D), jnp.float32),
        ],
    )(x, dp, s1, y1, b1, W2)


def _l2_body(h_ref, dp_ref, s_ref, y2_ref, b_ref, o_ref):
    dinv = _dinv_of(dp_ref)
    agg = s_ref[0] + s_ref[1] + y2_ref[...]
    o_ref[...] = h_ref[...] + dinv * agg + b_ref[...]


def _tc_layer2(h, dp, s2, y2, b2):
    return pl.pallas_call(
        _l2_body,
        grid=(N // BM,),
        in_specs=[
            pl.BlockSpec((BM, D), lambda i: (i, 0)),
            pl.BlockSpec((2, BM, 16), lambda i: (0, i, 0)),
            pl.BlockSpec((2, BM, D), lambda i: (0, i, 0)),
            pl.BlockSpec((BM, D), lambda i: (i, 0)),
            pl.BlockSpec((1, D), lambda i: (0, 0)),
        ],
        out_specs=pl.BlockSpec((BM, D), lambda i: (i, 0)),
        out_shape=jax.ShapeDtypeStruct((N, D), jnp.float32),
    )(h, dp, s2, y2, b2)


def kernel(x, edge_index, W1, b1, W2, b2):
    ei = edge_index.astype(jnp.int32)
    pad = EP - E
    row = jnp.concatenate(
        [ei[0], jnp.zeros((pad,), jnp.int32)]).reshape(NW, NCH, C)
    col = jnp.concatenate(
        [ei[1], jnp.full((pad,), N, jnp.int32)]).reshape(NW, NCH, C)
    b1r = b1.reshape(1, D)
    b2r = b2.reshape(1, D)

    dp = _sc_degree(col).reshape(NC, N, 16)
    xw1 = _tc_matmul(x, W1)
    y1 = _tc_norm(dp, xw1)
    s1 = _sc_scatter(y1, row, col).reshape(NC, N, D)
    h, y2 = _tc_layer1(x, dp, s1, y1, b1r, W2)
    s2 = _sc_scatter(y2, row, col).reshape(NC, N, D)
    return _tc_layer2(h, dp, s2, y2, b2r)
